# baseline (speedup 1.0000x reference)
import functools

import jax
import jax.numpy as jnp
from jax import lax
from jax.experimental import pallas as pl
from jax.experimental.pallas import tpu as pltpu
from jax.experimental.pallas import tpu_sc as plsc

_B = 16384
_CTX = 20
_D = 64
_DI = _D // 2
_TOT = 2 * _B
_NC, _NS = 2, 16
_NW = _NC * _NS
_PER_W = _TOT // _NW
_G = 128
_E = 32
_GPC = _E * _CTX // _G
_CHUNKS = _PER_W // _E
_LANES = 16
_ROWS = 199999


def _pack_tables(u_table, w_table):
    ub = jnp.pad(u_table, ((0, 1), (0, 0))).astype(jnp.bfloat16)
    wb = jnp.pad(w_table, ((0, 1), (0, 0))).astype(jnp.bfloat16)
    iu = lax.bitcast_convert_type(
        ub.reshape(_ROWS + 1, _DI, 2), jnp.int32)
    iw = lax.bitcast_convert_type(
        wb.reshape(_ROWS + 1, _DI, 2), jnp.int32)
    comb = jnp.concatenate(
        [iu[0::2], iw[0::2], iu[1::2], iw[1::2]], axis=1)
    return comb.reshape(2 * (_ROWS + 1), _DI)


def _prep_indices(pos_u, pos_w, neg_u, neg_w):
    u_idx = (2 * jnp.concatenate(
        [pos_u.reshape(-1), neg_u.reshape(-1)]
    ).astype(jnp.int32)).reshape(_NW, _CHUNKS * _GPC, _G)
    w_idx = (2 * jnp.concatenate([pos_w, neg_w]).astype(jnp.int32)
             + 1).reshape(_NW, _CHUNKS, _E)
    return u_idx, w_idx


def _sc_gather_sum(u_idx, w_idx, tab4):
    mesh = plsc.VectorSubcoreMesh(core_axis_name="c", subcore_axis_name="s")

    @functools.partial(
        pl.kernel,
        compiler_params=pltpu.CompilerParams(
            use_tc_tiling_on_sc=False, needs_layout_passes=False),
        out_type=jax.ShapeDtypeStruct((_TOT // 2, 4 * _DI), jnp.int32),
        mesh=mesh,
        scratch_types=[
            pltpu.VMEM((_CHUNKS * _GPC, _G), jnp.int32),
            pltpu.VMEM((_CHUNKS, _E), jnp.int32),
            pltpu.VMEM((_E * _CTX, _DI), jnp.int32),
            pltpu.VMEM((_E * _CTX, _DI), jnp.int32),
            pltpu.VMEM((_E, _DI), jnp.int32),
            pltpu.VMEM((_E, _DI), jnp.int32),
            pltpu.VMEM((_E // 2, 4 * _DI), jnp.int32),
            pltpu.VMEM((_E // 2, 4 * _DI), jnp.int32),
            pltpu.SemaphoreType.DMA,
            pltpu.SemaphoreType.DMA,
            pltpu.SemaphoreType.DMA,
            pltpu.SemaphoreType.DMA,
        ],
    )
    def k(uidx_hbm, widx_hbm, tab_hbm, out_hbm,
          uidx_v, widx_v, rows0, rows1, wrows0, wrows1, out0, out1,
          semg0, semg1, semo0, semo1):
        wid = lax.axis_index("s") * _NC + lax.axis_index("c")
        base2 = wid * _PER_W // 2
        pltpu.sync_copy(uidx_hbm.at[wid], uidx_v)
        pltpu.sync_copy(widx_hbm.at[wid], widx_v)

        def issue(ck, rows_v, wrows_v, semg):
            for j in range(_GPC):
                pltpu.async_copy(
                    tab_hbm.at[uidx_v.at[ck * _GPC + j]],
                    rows_v.at[pl.ds(j * _G, _G)],
                    semg,
                )
            pltpu.async_copy(tab_hbm.at[widx_v.at[ck]], wrows_v, semg)

        def drain(rows_v, wrows_v, semg):
            pltpu.make_async_copy(
                tab_hbm.at[pl.ds(0, _E * _CTX)], rows_v, semg).wait()
            pltpu.make_async_copy(tab_hbm.at[pl.ds(0, _E)], wrows_v, semg).wait()

        def compute(rows_v, wrows_v, out_v):
            @pl.loop(0, _E // 2)
            def _pair(p):
                for par in range(2):
                    r0 = (2 * p + par) * _CTX
                    ob = par * 2 * _DI
                    for h in range(_DI // _LANES):
                        sl = pl.ds(h * _LANES, _LANES)
                        acc = plsc.bitcast(rows_v[r0, sl], jnp.bfloat16)
                        for c in range(1, _CTX):
                            acc = acc + plsc.bitcast(
                                rows_v[r0 + c, sl], jnp.bfloat16)
                        out_v[p, pl.ds(ob + h * _LANES, _LANES)] = (
                            plsc.bitcast(acc, jnp.int32))
                        out_v[p, pl.ds(ob + _DI + h * _LANES, _LANES)] = (
                            wrows_v[2 * p + par, sl])

        def out_wait(out_v, semo):
            pltpu.make_async_copy(
                out_v, out_hbm.at[pl.ds(0, _E // 2)], semo).wait()

        _H = _CHUNKS // 2
        issue(0, rows0, wrows0, semg0)

        @pl.loop(0, _H)
        def _pipe(kk):
            ck0 = 2 * kk
            issue(ck0 + 1, rows1, wrows1, semg1)
            drain(rows0, wrows0, semg0)

            @pl.when(kk > 0)
            def _():
                out_wait(out0, semo0)

            compute(rows0, wrows0, out0)
            pltpu.async_copy(
                out0, out_hbm.at[pl.ds(base2 + ck0 * _E // 2, _E // 2)], semo0)

            @pl.when(kk < _H - 1)
            def _():
                issue(ck0 + 2, rows0, wrows0, semg0)

            drain(rows1, wrows1, semg1)

            @pl.when(kk > 0)
            def _():
                out_wait(out1, semo1)

            compute(rows1, wrows1, out1)
            pltpu.async_copy(
                out1, out_hbm.at[pl.ds(base2 + (ck0 + 1) * _E // 2, _E // 2)],
                semo1)

        out_wait(out0, semo0)
        out_wait(out1, semo1)

    return k(u_idx, w_idx, tab4)


def _tc_loss(packed):
    def body(x_ref, o_ref):
        x = x_ref[...]
        ev = lax.bitcast_convert_type(x << 16, jnp.float32)
        od = lax.bitcast_convert_type(
            x & jnp.int32(-65536), jnp.float32)
        s0 = jnp.sum(ev[:, :_DI] * ev[:, _DI:2 * _DI]
                     + od[:, :_DI] * od[:, _DI:2 * _DI],
                     axis=1, keepdims=True)
        s1 = jnp.sum(ev[:, 2 * _DI:3 * _DI] * ev[:, 3 * _DI:]
                     + od[:, 2 * _DI:3 * _DI] * od[:, 3 * _DI:],
                     axis=1, keepdims=True)
        row = lax.broadcasted_iota(jnp.int32, (_TOT // 2, 1), 0)
        sgn = jnp.where(row < _B // 2, -1.0, 1.0)
        ls = jax.nn.log_sigmoid(sgn * s0) + jax.nn.log_sigmoid(sgn * s1)
        o_ref[...] = jnp.sum(ls).reshape(1, 1)

    return pl.pallas_call(
        body,
        out_shape=jax.ShapeDtypeStruct((1, 1), jnp.float32),
    )(packed)


def kernel(pos_u, pos_w, neg_u, neg_w, n, u_table, w_table):
    u_idx, w_idx = _prep_indices(pos_u, pos_w, neg_u, neg_w)
    tab4 = _pack_tables(u_table, w_table)
    packed = _sc_gather_sum(u_idx, w_idx, tab4)
    loss = _tc_loss(packed)[0, 0]
    return -1.0 * loss / n
